# baseline (speedup 1.0000x reference)
import jax
import jax.numpy as jnp
from jax import lax
from jax.experimental import pallas as pl
from jax.experimental.pallas import tpu as pltpu
from jax.experimental.pallas import tpu_sc as plsc

N = 10000
E = 320000
D = 128
H = 128
C = 16
EPS = 1e-05

NC = 2
NS = 16
NW = NC * NS
EPW = E // NW
CHUNK = 80
NCHUNK = EPW // CHUNK
NCHUNKP = 128
NP = 10240
RPT = NP // NS
GPW = 2048 // NW

_mesh = plsc.VectorSubcoreMesh(core_axis_name="c", subcore_axis_name="s",
                               num_cores=NC, num_subcores=NS)


def _make_spmm_body(gather_out):
    def _spmm_body(h_hbm, src3_hbm, dst_hbm, w_hbm, idx_hbm,
                   out0_hbm, out1_hbm,
                   acc_sh, src_all, dstb, wbuf, rows0, rows1, rows2,
                   semg0, semg1, semg2, sems0, sems1, sems2,
                   semi0, semi1, semi2):
        c = lax.axis_index("c")
        s = lax.axis_index("s")
        wid = c * NS + s
        e_base = pl.multiple_of(wid * EPW, 8)

        pltpu.sync_copy(src3_hbm.at[wid], src_all)

        def _zb(i, _):
            for j in range(D // 16):
                rows0[i, pl.ds(j * 16, 16)] = jnp.zeros((16,), jnp.float32)
            return _
        lax.fori_loop(0, CHUNK, _zb, None)
        r0 = s * RPT
        for k in range(RPT // CHUNK):
            pltpu.async_copy(rows0, acc_sh.at[pl.ds(r0 + k * CHUNK, CHUNK)],
                             semg0)
        for k in range(RPT // CHUNK):
            pltpu.make_async_copy(rows0, acc_sh.at[pl.ds(r0, CHUNK)],
                                  semg0).wait()
        plsc.subcore_barrier()

        bufs = (rows0, rows1, rows2)
        semg = (semg0, semg1, semg2)
        semsc = (sems0, sems1, sems2)
        semi = (semi0, semi1, semi2)

        def iidx(ch, b):
            e0 = pl.multiple_of(e_base + ch * CHUNK, 8)
            pltpu.async_copy(dst_hbm.at[pl.ds(e0, CHUNK)], dstb.at[b],
                             semi[b])
            pltpu.async_copy(w_hbm.at[pl.ds(e0, CHUNK)], wbuf.at[b], semi[b])

        def widx(b):
            pltpu.make_async_copy(dst_hbm.at[pl.ds(0, CHUNK)], dstb.at[b],
                                  semi[b]).wait()
            pltpu.make_async_copy(w_hbm.at[pl.ds(0, CHUNK)], wbuf.at[b],
                                  semi[b]).wait()

        def igather(ch, b):
            pltpu.async_copy(h_hbm.at[src_all.at[ch]], bufs[b], semg[b])

        def wgather(b):
            pltpu.make_async_copy(h_hbm.at[src_all.at[0]], bufs[b],
                                  semg[b]).wait()

        def iscat(b):
            pltpu.async_copy(bufs[b], acc_sh.at[dstb.at[b]], semsc[b],
                             add=True)

        def wscat(b):
            pltpu.make_async_copy(bufs[b], acc_sh.at[dstb.at[b]],
                                  semsc[b]).wait()

        def scale(b, glo, ghi):
            rows = bufs[b]

            def _sc(g, _):
                wv = wbuf[b, pl.ds(g * 16, 16)]
                for k in range(16):
                    i = g * 16 + k
                    wb = wv[k]
                    for j in range(D // 16):
                        rows[i, pl.ds(j * 16, 16)] = (
                            rows[i, pl.ds(j * 16, 16)] * wb)
                return _
            lax.fori_loop(glo, ghi, _sc, None)

        def proc(ch, a, issue):
            p = (a + 2) % 3
            wgather(a)
            widx(a)
            scale(a, 0, 1)
            if issue:
                @pl.when(ch >= 1)
                def _():
                    wscat(p)
                iidx(ch + 2, p)
                igather(ch + 2, p)
            else:
                wscat(p)
            scale(a, 1, CHUNK // 16)
            iscat(a)

        iidx(0, 0)
        igather(0, 0)
        iidx(1, 1)
        igather(1, 1)

        def _trip(t, _):
            ch = t * 3
            proc(ch, 0, True)
            proc(ch + 1, 1, True)
            proc(ch + 2, 2, True)
            return _
        lax.fori_loop(0, (NCHUNK - 2) // 3, _trip, None)
        proc(NCHUNK - 2, 0, False)
        proc(NCHUNK - 1, 1, False)
        wscat(1)
        plsc.subcore_barrier()

        if gather_out:
            for k in range(2):
                base = pl.multiple_of((s * 2 + k) * GPW, 8)
                pltpu.sync_copy(idx_hbm.at[pl.ds(base, GPW)],
                                dstb.at[0, pl.ds(0, GPW)])
                pltpu.async_copy(acc_sh.at[dstb.at[0, pl.ds(0, GPW)]],
                                 rows0.at[pl.ds(0, GPW)], semg0).wait()

                @pl.when(c == 0)
                def _():
                    pltpu.sync_copy(rows0.at[pl.ds(0, GPW)],
                                    out0_hbm.at[pl.ds(base, GPW)])

                @pl.when(c == 1)
                def _():
                    pltpu.sync_copy(rows0.at[pl.ds(0, GPW)],
                                    out1_hbm.at[pl.ds(base, GPW)])
        else:
            nko = RPT // CHUNK
            obufs = (rows0, rows1)
            osems = (semg0, semg1)
            pltpu.async_copy(acc_sh.at[pl.ds(r0, CHUNK)], rows0, semg0)
            for k in range(nko):
                b = k % 2
                rr = r0 + k * CHUNK
                pltpu.make_async_copy(acc_sh.at[pl.ds(rr, CHUNK)], obufs[b],
                                      osems[b]).wait()
                if k + 1 < nko:
                    pltpu.async_copy(
                        acc_sh.at[pl.ds(rr + CHUNK, CHUNK)],
                        obufs[1 - b], osems[1 - b])

                @pl.when(c == 0)
                def _():
                    pltpu.sync_copy(obufs[b], out0_hbm.at[pl.ds(rr, CHUNK)])

                @pl.when(c == 1)
                def _():
                    pltpu.sync_copy(obufs[b], out1_hbm.at[pl.ds(rr, CHUNK)])

    return _spmm_body


def _spmm_scratch():
    return [
        pltpu.VMEM_SHARED((NP, D), jnp.float32),
        pltpu.VMEM((NCHUNKP, CHUNK), jnp.int32),
        pltpu.VMEM((3, CHUNK), jnp.int32),
        pltpu.VMEM((3, CHUNK), jnp.float32),
        pltpu.VMEM((CHUNK, D), jnp.float32),
        pltpu.VMEM((CHUNK, D), jnp.float32),
        pltpu.VMEM((CHUNK, D), jnp.float32),
        pltpu.SemaphoreType.DMA,
        pltpu.SemaphoreType.DMA,
        pltpu.SemaphoreType.DMA,
        pltpu.SemaphoreType.DMA,
        pltpu.SemaphoreType.DMA,
        pltpu.SemaphoreType.DMA,
        pltpu.SemaphoreType.DMA,
        pltpu.SemaphoreType.DMA,
        pltpu.SemaphoreType.DMA,
    ]


_spmm = pl.kernel(
    _make_spmm_body(False),
    out_type=(jax.ShapeDtypeStruct((NP, D), jnp.float32),
              jax.ShapeDtypeStruct((NP, D), jnp.float32)),
    mesh=_mesh,
    scratch_types=_spmm_scratch(),
)

_spmm_g = pl.kernel(
    _make_spmm_body(True),
    out_type=(jax.ShapeDtypeStruct((2048, D), jnp.float32),
              jax.ShapeDtypeStruct((2048, D), jnp.float32)),
    mesh=_mesh,
    scratch_types=_spmm_scratch(),
)

BM = 1000
BMP = 1024


def _inproj_body(x_ref, w_ref, b_ref, o_ref):
    o_ref[...] = jnp.dot(x_ref[...], w_ref[...],
                         preferred_element_type=jnp.float32) + b_ref[...]


def _tc_inproj(x, w, b):
    return pl.pallas_call(
        _inproj_body,
        grid=(N // BM,),
        in_specs=[pl.BlockSpec((BM, D), lambda i: (i, 0)),
                  pl.BlockSpec((D, H), lambda i: (0, 0)),
                  pl.BlockSpec((1, H), lambda i: (0, 0))],
        out_specs=pl.BlockSpec((BM, H), lambda i: (i, 0)),
        out_shape=jax.ShapeDtypeStruct((N, H), jnp.float32),
    )(x, w, b)


def _mid_body(p0_ref, p1_ref, a_ref, cc_ref, w_ref, o_ref):
    h = jnp.maximum((p0_ref[...] + p1_ref[...]) * a_ref[...] + cc_ref[...], 0.0)
    o_ref[...] = jnp.dot(h, w_ref[...], preferred_element_type=jnp.float32)


def _tc_mid(p0, p1, a, cc, w):
    return pl.pallas_call(
        _mid_body,
        grid=(NP // BMP,),
        in_specs=[pl.BlockSpec((BMP, H), lambda i: (i, 0)),
                  pl.BlockSpec((BMP, H), lambda i: (i, 0)),
                  pl.BlockSpec((1, H), lambda i: (0, 0)),
                  pl.BlockSpec((1, H), lambda i: (0, 0)),
                  pl.BlockSpec((H, H), lambda i: (0, 0))],
        out_specs=pl.BlockSpec((BMP, H), lambda i: (i, 0)),
        out_shape=jax.ShapeDtypeStruct((NP, H), jnp.float32),
    )(p0, p1, a, cc, w)


def _head_body(g0_ref, g1_ref, a_ref, cc_ref, w_ref, b_ref, o_ref):
    h = jnp.maximum((g0_ref[...] + g1_ref[...]) * a_ref[...] + cc_ref[...], 0.0)
    o_ref[...] = (jnp.dot(h, w_ref[...], preferred_element_type=jnp.float32)
                  + b_ref[...])


def _tc_head(g0, g1, a, cc, w, b):
    return pl.pallas_call(
        _head_body,
        in_specs=[pl.BlockSpec((2048, H), lambda: (0, 0)),
                  pl.BlockSpec((2048, H), lambda: (0, 0)),
                  pl.BlockSpec((1, H), lambda: (0, 0)),
                  pl.BlockSpec((1, H), lambda: (0, 0)),
                  pl.BlockSpec((H, C), lambda: (0, 0)),
                  pl.BlockSpec((1, C), lambda: (0, 0))],
        out_specs=pl.BlockSpec((2048, C), lambda: (0, 0)),
        out_shape=jax.ShapeDtypeStruct((2048, C), jnp.float32),
    )(g0, g1, a, cc, w, b)


def kernel(features, edge_index, edge_weight, idx, W0, b0, bias0, gamma0,
           beta0, mean0, var0, W1, bias1, gamma1, beta1, mean1, var1, Wf, bf):
    src = edge_index[0]
    dst = edge_index[1]

    a0 = gamma0 * lax.rsqrt(var0 + EPS)
    c0 = (bias0 - mean0) * a0 + beta0
    a1 = gamma1 * lax.rsqrt(var1 + EPS)
    c1 = (bias1 - mean1) * a1 + beta1

    src3 = jnp.pad(src.reshape(NW, NCHUNK, CHUNK),
                   ((0, 0), (0, NCHUNKP - NCHUNK), (0, 0)))

    h0 = _tc_inproj(features, W0, b0.reshape(1, H))
    p0, p1 = _spmm(h0, src3, dst, edge_weight, idx)
    h1 = _tc_mid(p0, p1, a0.reshape(1, H), c0.reshape(1, H), W1)
    g0, g1 = _spmm_g(h1, src3, dst, edge_weight, idx)
    return _tc_head(g0, g1, a1.reshape(1, H), c1.reshape(1, H), Wf,
                    bf.reshape(1, C))


# speedup vs baseline: 1.0083x; 1.0083x over previous
import jax
import jax.numpy as jnp
from jax import lax
from jax.experimental import pallas as pl
from jax.experimental.pallas import tpu as pltpu
from jax.experimental.pallas import tpu_sc as plsc

N = 10000
E = 320000
D = 128
H = 128
C = 16
EPS = 1e-05

NC = 2
NS = 16
NW = NC * NS
EPW = E // NW
CHUNK = 80
NCHUNK = EPW // CHUNK
NCHUNKP = 128
NP = 10240
RPT = NP // NS
GPW = 2048 // NW

_mesh = plsc.VectorSubcoreMesh(core_axis_name="c", subcore_axis_name="s",
                               num_cores=NC, num_subcores=NS)


def _make_spmm_body(gather_out):
    def _spmm_body(h_hbm, src3_hbm, dst_hbm, w_hbm, idx_hbm,
                   out0_hbm, out1_hbm,
                   acc_sh, src_all, dstb, wbuf, rows0, rows1, rows2,
                   semg0, semg1, semg2, sems0, sems1, sems2,
                   semi0, semi1, semi2):
        c = lax.axis_index("c")
        s = lax.axis_index("s")
        wid = c * NS + s
        e_base = pl.multiple_of(wid * EPW, 8)

        pltpu.sync_copy(src3_hbm.at[wid], src_all)

        def _zb(i, _):
            for j in range(D // 16):
                rows0[i, pl.ds(j * 16, 16)] = jnp.zeros((16,), jnp.float32)
            return _
        lax.fori_loop(0, CHUNK, _zb, None)
        r0 = s * RPT
        for k in range(RPT // CHUNK):
            pltpu.async_copy(rows0, acc_sh.at[pl.ds(r0 + k * CHUNK, CHUNK)],
                             semg0)
        for k in range(RPT // CHUNK):
            pltpu.make_async_copy(rows0, acc_sh.at[pl.ds(r0, CHUNK)],
                                  semg0).wait()
        plsc.subcore_barrier()

        bufs = (rows0, rows1, rows2)
        semg = (semg0, semg1, semg2)
        semsc = (sems0, sems1, sems2)
        semi = (semi0, semi1, semi2)

        def iidx(ch, b):
            e0 = pl.multiple_of(e_base + ch * CHUNK, 8)
            pltpu.async_copy(dst_hbm.at[pl.ds(e0, CHUNK)], dstb.at[b],
                             semi[b])
            pltpu.async_copy(w_hbm.at[pl.ds(e0, CHUNK)], wbuf.at[b], semi[b])

        def widx(b):
            pltpu.make_async_copy(dst_hbm.at[pl.ds(0, CHUNK)], dstb.at[b],
                                  semi[b]).wait()
            pltpu.make_async_copy(w_hbm.at[pl.ds(0, CHUNK)], wbuf.at[b],
                                  semi[b]).wait()

        def igather(ch, b):
            pltpu.async_copy(h_hbm.at[src_all.at[ch]], bufs[b], semg[b])

        def wgather(b):
            pltpu.make_async_copy(h_hbm.at[src_all.at[0]], bufs[b],
                                  semg[b]).wait()

        def iscat(b):
            pltpu.async_copy(bufs[b], acc_sh.at[dstb.at[b]], semsc[b],
                             add=True)

        def wscat(b):
            pltpu.make_async_copy(bufs[b], acc_sh.at[dstb.at[b]],
                                  semsc[b]).wait()

        def scale(b, glo, ghi):
            rows = bufs[b]

            def _sc(g, _):
                wv = wbuf[b, pl.ds(g * 16, 16)]
                for k in range(16):
                    i = g * 16 + k
                    wb = wv[k]
                    for j in range(D // 16):
                        rows[i, pl.ds(j * 16, 16)] = (
                            rows[i, pl.ds(j * 16, 16)] * wb)
                return _
            lax.fori_loop(glo, ghi, _sc, None)

        def proc(ch, a, issue):
            p = (a + 2) % 3
            wgather(a)
            widx(a)
            scale(a, 0, 2)
            if issue:
                @pl.when(ch >= 1)
                def _():
                    wscat(p)
                iidx(ch + 2, p)
                igather(ch + 2, p)
            else:
                wscat(p)
            scale(a, 2, CHUNK // 16)
            iscat(a)

        iidx(0, 0)
        igather(0, 0)
        iidx(1, 1)
        igather(1, 1)

        def _trip(t, _):
            ch = t * 3
            proc(ch, 0, True)
            proc(ch + 1, 1, True)
            proc(ch + 2, 2, True)
            return _
        lax.fori_loop(0, (NCHUNK - 2) // 3, _trip, None)
        proc(NCHUNK - 2, 0, False)
        proc(NCHUNK - 1, 1, False)
        wscat(1)
        plsc.subcore_barrier()

        if gather_out:
            for k in range(2):
                base = pl.multiple_of((s * 2 + k) * GPW, 8)
                pltpu.sync_copy(idx_hbm.at[pl.ds(base, GPW)],
                                dstb.at[0, pl.ds(0, GPW)])
                pltpu.async_copy(acc_sh.at[dstb.at[0, pl.ds(0, GPW)]],
                                 rows0.at[pl.ds(0, GPW)], semg0).wait()

                @pl.when(c == 0)
                def _():
                    pltpu.sync_copy(rows0.at[pl.ds(0, GPW)],
                                    out0_hbm.at[pl.ds(base, GPW)])

                @pl.when(c == 1)
                def _():
                    pltpu.sync_copy(rows0.at[pl.ds(0, GPW)],
                                    out1_hbm.at[pl.ds(base, GPW)])
        else:
            nko = RPT // CHUNK
            obufs = (rows0, rows1)
            osems = (semg0, semg1)
            pltpu.async_copy(acc_sh.at[pl.ds(r0, CHUNK)], rows0, semg0)
            for k in range(nko):
                b = k % 2
                rr = r0 + k * CHUNK
                pltpu.make_async_copy(acc_sh.at[pl.ds(rr, CHUNK)], obufs[b],
                                      osems[b]).wait()
                if k + 1 < nko:
                    pltpu.async_copy(
                        acc_sh.at[pl.ds(rr + CHUNK, CHUNK)],
                        obufs[1 - b], osems[1 - b])

                @pl.when(c == 0)
                def _():
                    pltpu.sync_copy(obufs[b], out0_hbm.at[pl.ds(rr, CHUNK)])

                @pl.when(c == 1)
                def _():
                    pltpu.sync_copy(obufs[b], out1_hbm.at[pl.ds(rr, CHUNK)])

    return _spmm_body


def _spmm_scratch():
    return [
        pltpu.VMEM_SHARED((NP, D), jnp.float32),
        pltpu.VMEM((NCHUNKP, CHUNK), jnp.int32),
        pltpu.VMEM((3, CHUNK), jnp.int32),
        pltpu.VMEM((3, CHUNK), jnp.float32),
        pltpu.VMEM((CHUNK, D), jnp.float32),
        pltpu.VMEM((CHUNK, D), jnp.float32),
        pltpu.VMEM((CHUNK, D), jnp.float32),
        pltpu.SemaphoreType.DMA,
        pltpu.SemaphoreType.DMA,
        pltpu.SemaphoreType.DMA,
        pltpu.SemaphoreType.DMA,
        pltpu.SemaphoreType.DMA,
        pltpu.SemaphoreType.DMA,
        pltpu.SemaphoreType.DMA,
        pltpu.SemaphoreType.DMA,
        pltpu.SemaphoreType.DMA,
    ]


_spmm = pl.kernel(
    _make_spmm_body(False),
    out_type=(jax.ShapeDtypeStruct((NP, D), jnp.float32),
              jax.ShapeDtypeStruct((NP, D), jnp.float32)),
    mesh=_mesh,
    scratch_types=_spmm_scratch(),
)

_spmm_g = pl.kernel(
    _make_spmm_body(True),
    out_type=(jax.ShapeDtypeStruct((2048, D), jnp.float32),
              jax.ShapeDtypeStruct((2048, D), jnp.float32)),
    mesh=_mesh,
    scratch_types=_spmm_scratch(),
)

BM = 1000
BMP = 1024


def _inproj_body(x_ref, w_ref, b_ref, o_ref):
    o_ref[...] = jnp.dot(x_ref[...], w_ref[...],
                         preferred_element_type=jnp.float32) + b_ref[...]


def _tc_inproj(x, w, b):
    return pl.pallas_call(
        _inproj_body,
        grid=(N // BM,),
        in_specs=[pl.BlockSpec((BM, D), lambda i: (i, 0)),
                  pl.BlockSpec((D, H), lambda i: (0, 0)),
                  pl.BlockSpec((1, H), lambda i: (0, 0))],
        out_specs=pl.BlockSpec((BM, H), lambda i: (i, 0)),
        out_shape=jax.ShapeDtypeStruct((N, H), jnp.float32),
    )(x, w, b)


def _mid_body(p0_ref, p1_ref, a_ref, cc_ref, w_ref, o_ref):
    h = jnp.maximum((p0_ref[...] + p1_ref[...]) * a_ref[...] + cc_ref[...], 0.0)
    o_ref[...] = jnp.dot(h, w_ref[...], preferred_element_type=jnp.float32)


def _tc_mid(p0, p1, a, cc, w):
    return pl.pallas_call(
        _mid_body,
        grid=(NP // BMP,),
        in_specs=[pl.BlockSpec((BMP, H), lambda i: (i, 0)),
                  pl.BlockSpec((BMP, H), lambda i: (i, 0)),
                  pl.BlockSpec((1, H), lambda i: (0, 0)),
                  pl.BlockSpec((1, H), lambda i: (0, 0)),
                  pl.BlockSpec((H, H), lambda i: (0, 0))],
        out_specs=pl.BlockSpec((BMP, H), lambda i: (i, 0)),
        out_shape=jax.ShapeDtypeStruct((NP, H), jnp.float32),
    )(p0, p1, a, cc, w)


def _head_body(g0_ref, g1_ref, a_ref, cc_ref, w_ref, b_ref, o_ref):
    h = jnp.maximum((g0_ref[...] + g1_ref[...]) * a_ref[...] + cc_ref[...], 0.0)
    o_ref[...] = (jnp.dot(h, w_ref[...], preferred_element_type=jnp.float32)
                  + b_ref[...])


def _tc_head(g0, g1, a, cc, w, b):
    return pl.pallas_call(
        _head_body,
        in_specs=[pl.BlockSpec((2048, H), lambda: (0, 0)),
                  pl.BlockSpec((2048, H), lambda: (0, 0)),
                  pl.BlockSpec((1, H), lambda: (0, 0)),
                  pl.BlockSpec((1, H), lambda: (0, 0)),
                  pl.BlockSpec((H, C), lambda: (0, 0)),
                  pl.BlockSpec((1, C), lambda: (0, 0))],
        out_specs=pl.BlockSpec((2048, C), lambda: (0, 0)),
        out_shape=jax.ShapeDtypeStruct((2048, C), jnp.float32),
    )(g0, g1, a, cc, w, b)


def kernel(features, edge_index, edge_weight, idx, W0, b0, bias0, gamma0,
           beta0, mean0, var0, W1, bias1, gamma1, beta1, mean1, var1, Wf, bf):
    src = edge_index[0]
    dst = edge_index[1]

    a0 = gamma0 * lax.rsqrt(var0 + EPS)
    c0 = (bias0 - mean0) * a0 + beta0
    a1 = gamma1 * lax.rsqrt(var1 + EPS)
    c1 = (bias1 - mean1) * a1 + beta1

    src3 = jnp.pad(src.reshape(NW, NCHUNK, CHUNK),
                   ((0, 0), (0, NCHUNKP - NCHUNK), (0, 0)))

    h0 = _tc_inproj(features, W0, b0.reshape(1, H))
    p0, p1 = _spmm(h0, src3, dst, edge_weight, idx)
    h1 = _tc_mid(p0, p1, a0.reshape(1, H), c0.reshape(1, H), W1)
    g0, g1 = _spmm_g(h1, src3, dst, edge_weight, idx)
    return _tc_head(g0, g1, a1.reshape(1, H), c1.reshape(1, H), Wf,
                    bf.reshape(1, C))
